# baseline (speedup 1.0000x reference)
import functools

import jax
import jax.numpy as jnp
from jax import lax
from jax.experimental import pallas as pl
from jax.experimental.pallas import tpu as pltpu
from jax.experimental.pallas import tpu_sc as plsc

N = 10000
E = 320000
H = 128

NC = 2
NS = 16
NW = NC * NS
EPW = E // NW
WIN = 128
NF = EPW // WIN
TWIN = EPW - NF * WIN
NPAD = 10112
ZROWS = NPAD // NS
OROWS = 624
TAIL = N - NS * OROWS

_MESH = plsc.VectorSubcoreMesh(
    core_axis_name="c", subcore_axis_name="s", num_cores=NC, num_subcores=NS
)


def _zero_acc(zeros_hbm, acc, sid):
    base = sid * ZROWS
    for z in range(4):
        pltpu.sync_copy(zeros_hbm, acc.at[pl.ds(base + z * WIN, WIN)])
    pltpu.sync_copy(zeros_hbm.at[pl.ds(0, ZROWS - 4 * WIN)],
                    acc.at[pl.ds(base + 4 * WIN, ZROWS - 4 * WIN)])


@functools.partial(
    pl.kernel,
    out_type=jax.ShapeDtypeStruct((NC, N, H), jnp.float32),
    mesh=_MESH,
    scratch_types=[
        pltpu.VMEM((WIN,), jnp.int32),
        pltpu.VMEM((WIN,), jnp.int32),
        pltpu.VMEM((TWIN,), jnp.int32),
        pltpu.VMEM((WIN, H), jnp.float32),
        pltpu.VMEM_SHARED((NPAD, H), jnp.float32),
        pltpu.SemaphoreType.DMA,
        pltpu.SemaphoreType.DMA,
    ],
)
def _deg_kernel(dst_hbm, zeros_hbm, ones_hbm, out_hbm, dst_w0, dst_w1, dst_t,
                ones_v, acc, sem0, sem1):
    cid = lax.axis_index("c")
    sid = lax.axis_index("s")
    wid = cid * NS + sid
    dbuf = (dst_w0, dst_w1)
    sems = (sem0, sem1)
    descs = [None] * NF
    descs[0] = pltpu.async_copy(dst_hbm.at[wid, pl.ds(0, WIN)], dbuf[0], sems[0])
    pltpu.sync_copy(dst_hbm.at[wid, pl.ds(NF * WIN, TWIN)], dst_t)
    _zero_acc(zeros_hbm, acc, sid)
    pltpu.sync_copy(ones_hbm, ones_v)
    plsc.subcore_barrier()
    for w in range(NF):
        b = w % 2
        if w + 1 < NF:
            descs[w + 1] = pltpu.async_copy(
                dst_hbm.at[wid, pl.ds((w + 1) * WIN, WIN)], dbuf[1 - b], sems[1 - b]
            )
        descs[w].wait()
        pltpu.sync_copy(ones_v, acc.at[dbuf[b]], add=True)
    pltpu.sync_copy(ones_v.at[pl.ds(0, TWIN)], acc.at[dst_t], add=True)
    plsc.subcore_barrier()
    pltpu.sync_copy(
        acc.at[pl.ds(sid * OROWS, OROWS)], out_hbm.at[cid, pl.ds(sid * OROWS, OROWS)]
    )

    @pl.when(sid == NS - 1)
    def _tail():
        pltpu.sync_copy(
            acc.at[pl.ds(NS * OROWS, TAIL)], out_hbm.at[cid, pl.ds(NS * OROWS, TAIL)]
        )


@functools.partial(
    pl.kernel,
    out_type=jax.ShapeDtypeStruct((NC, N, H), jnp.float32),
    mesh=_MESH,
    scratch_types=[
        pltpu.VMEM((WIN,), jnp.int32),
        pltpu.VMEM((WIN,), jnp.int32),
        pltpu.VMEM((WIN,), jnp.int32),
        pltpu.VMEM((WIN,), jnp.int32),
        pltpu.VMEM((WIN,), jnp.int32),
        pltpu.VMEM((WIN,), jnp.int32),
        pltpu.VMEM((TWIN,), jnp.int32),
        pltpu.VMEM((TWIN,), jnp.int32),
        pltpu.VMEM((WIN, H), jnp.float32),
        pltpu.VMEM((WIN, H), jnp.float32),
        pltpu.VMEM_SHARED((NPAD, H), jnp.float32),
        pltpu.SemaphoreType.DMA,
        pltpu.SemaphoreType.DMA,
        pltpu.SemaphoreType.DMA,
        pltpu.SemaphoreType.DMA,
        pltpu.SemaphoreType.DMA,
        pltpu.SemaphoreType.DMA,
        pltpu.SemaphoreType.DMA,
        pltpu.SemaphoreType.DMA,
        pltpu.SemaphoreType.DMA,
        pltpu.SemaphoreType.DMA,
    ],
)
def _msg_kernel(hp_hbm, src_hbm, dst_hbm, zeros_hbm, out_hbm,
                src_w0, src_w1, src_w2, dst_w0, dst_w1, dst_w2, src_t, dst_t,
                rows0, rows1, acc, is0, is1, is2, js0, js1, js2,
                gs0, gs1, ss0, ss1):
    cid = lax.axis_index("c")
    sid = lax.axis_index("s")
    wid = cid * NS + sid
    sbuf = (src_w0, src_w1, src_w2)
    dbuf = (dst_w0, dst_w1, dst_w2)
    rbuf = (rows0, rows1)
    isems = (is0, is1, is2)
    jsems = (js0, js1, js2)
    gsems = (gs0, gs1)
    ssems = (ss0, ss1)
    si = [None] * NF
    di = [None] * NF
    gd = [None] * NF
    sc = [None] * NF
    for p in range(2):
        si[p] = pltpu.async_copy(
            src_hbm.at[wid, pl.ds(p * WIN, WIN)], sbuf[p], isems[p]
        )
        di[p] = pltpu.async_copy(
            dst_hbm.at[wid, pl.ds(p * WIN, WIN)], dbuf[p], jsems[p]
        )
    pltpu.sync_copy(src_hbm.at[wid, pl.ds(NF * WIN, TWIN)], src_t)
    pltpu.sync_copy(dst_hbm.at[wid, pl.ds(NF * WIN, TWIN)], dst_t)
    _zero_acc(zeros_hbm, acc, sid)
    plsc.subcore_barrier()
    si[0].wait()
    gd[0] = pltpu.async_copy(hp_hbm.at[sbuf[0]], rbuf[0], gsems[0])
    for w in range(NF):
        b = w % 2
        nb = 1 - b
        gd[w].wait()
        di[w].wait()
        sc[w] = pltpu.async_copy(rbuf[b], acc.at[dbuf[w % 3]], ssems[b],
                                 add=True)
        if w >= 1:
            sc[w - 1].wait()
        if w + 2 < NF:
            s3 = (w + 2) % 3
            si[w + 2] = pltpu.async_copy(
                src_hbm.at[wid, pl.ds((w + 2) * WIN, WIN)], sbuf[s3], isems[s3]
            )
            di[w + 2] = pltpu.async_copy(
                dst_hbm.at[wid, pl.ds((w + 2) * WIN, WIN)], dbuf[s3], jsems[s3]
            )
        if w + 1 < NF:
            si[w + 1].wait()
            gd[w + 1] = pltpu.async_copy(hp_hbm.at[sbuf[(w + 1) % 3]], rbuf[nb],
                                         gsems[nb])
    sc[NF - 1].wait()
    tb = NF % 2
    pltpu.async_copy(hp_hbm.at[src_t], rbuf[tb].at[pl.ds(0, TWIN)],
                     gsems[tb]).wait()
    pltpu.sync_copy(rbuf[tb].at[pl.ds(0, TWIN)], acc.at[dst_t], add=True)
    plsc.subcore_barrier()
    pltpu.sync_copy(
        acc.at[pl.ds(sid * OROWS, OROWS)], out_hbm.at[cid, pl.ds(sid * OROWS, OROWS)]
    )

    @pl.when(sid == NS - 1)
    def _tail():
        pltpu.sync_copy(
            acc.at[pl.ds(NS * OROWS, TAIL)], out_hbm.at[cid, pl.ds(NS * OROWS, TAIL)]
        )


def _prep_body(dp_ref, x_ref, w0_ref, dinv_ref, hp_ref):
    deg = dp_ref[0][:, 0:1] + dp_ref[1][:, 0:1] + 1.0
    dinv = lax.rsqrt(deg)
    dinv_ref[...] = dinv
    h = jnp.dot(x_ref[...], w0_ref[...], preferred_element_type=jnp.float32,
                precision=lax.Precision.HIGHEST)
    hp_ref[...] = h * dinv


def _bn_body(agg_ref, hp_ref, dinv_ref, b_ref, g_ref, be_ref, yprev_ref,
             y_ref, *, has_resid):
    z = (agg_ref[0] + agg_ref[1] + hp_ref[...]) * dinv_ref[...] + b_ref[...]
    if has_resid:
        z = z + yprev_ref[...]
    mu = jnp.mean(z, axis=0, keepdims=True)
    d = z - mu
    var = jnp.mean(d * d, axis=0, keepdims=True)
    y = g_ref[...] * d * lax.rsqrt(var + 1e-5) + be_ref[...]
    y_ref[...] = jnp.maximum(y, 0.0)


def _mm_body(y_ref, wn_ref, dinv_ref, hn_ref):
    hn_ref[...] = jnp.dot(y_ref[...], wn_ref[...],
                          preferred_element_type=jnp.float32,
                          precision=lax.Precision.HIGHEST) * dinv_ref[...]


_F = jnp.float32
_prep_call = pl.pallas_call(
    _prep_body,
    out_shape=(jax.ShapeDtypeStruct((N, 1), _F), jax.ShapeDtypeStruct((N, H), _F)))
_bn0_call = pl.pallas_call(
    functools.partial(_bn_body, has_resid=False),
    out_shape=jax.ShapeDtypeStruct((N, H), _F))
_bn_resid_call = pl.pallas_call(
    functools.partial(_bn_body, has_resid=True),
    out_shape=jax.ShapeDtypeStruct((N, H), _F))
_mm_call = pl.pallas_call(
    _mm_body, out_shape=jax.ShapeDtypeStruct((N, H), _F))


def kernel(x, edge_index, W0, b0, g0, be0, W1, b1, g1, be1, W2, b2, g2, be2):
    src_w = edge_index[0].reshape(NW, EPW)
    dst_w = edge_index[1].reshape(NW, EPW)

    zeros_hbm = jnp.zeros((WIN, H), _F)
    ones_hbm = jnp.ones((WIN, H), _F)

    b0r, g0r, be0r = b0.reshape(1, H), g0.reshape(1, H), be0.reshape(1, H)
    b1r, g1r, be1r = b1.reshape(1, H), g1.reshape(1, H), be1.reshape(1, H)
    b2r, g2r, be2r = b2.reshape(1, H), g2.reshape(1, H), be2.reshape(1, H)

    dp = _deg_kernel(dst_w, zeros_hbm, ones_hbm)
    dinv, h0p = _prep_call(dp, x, W0)
    agg0 = _msg_kernel(h0p, src_w, dst_w, zeros_hbm)
    y0 = _bn0_call(agg0, h0p, dinv, b0r, g0r, be0r, h0p)
    h1p = _mm_call(y0, W1, dinv)
    agg1 = _msg_kernel(h1p, src_w, dst_w, zeros_hbm)
    y1 = _bn_resid_call(agg1, h1p, dinv, b1r, g1r, be1r, y0)
    h2p = _mm_call(y1, W2, dinv)
    agg2 = _msg_kernel(h2p, src_w, dst_w, zeros_hbm)
    y2 = _bn_resid_call(agg2, h2p, dinv, b2r, g2r, be2r, y1)
    return y2


# speedup vs baseline: 1.0361x; 1.0361x over previous
import functools

import jax
import jax.numpy as jnp
from jax import lax
from jax.experimental import pallas as pl
from jax.experimental.pallas import tpu as pltpu
from jax.experimental.pallas import tpu_sc as plsc

N = 10000
E = 320000
H = 128

NC = 2
NS = 16
NW = NC * NS
EPW = E // NW
WIN = 128
NF = EPW // WIN
TWIN = EPW - NF * WIN
NPAD = 10112
ZROWS = NPAD // NS
OROWS = 624
TAIL = N - NS * OROWS

_MESH = plsc.VectorSubcoreMesh(
    core_axis_name="c", subcore_axis_name="s", num_cores=NC, num_subcores=NS
)


def _zero_acc(zeros_hbm, acc, sid):
    pltpu.sync_copy(
        zeros_hbm.at[pl.ds(sid * ZROWS, ZROWS)], acc.at[pl.ds(sid * ZROWS, ZROWS)]
    )


@functools.partial(
    pl.kernel,
    out_type=jax.ShapeDtypeStruct((NC, N, H), jnp.float32),
    mesh=_MESH,
    scratch_types=[
        pltpu.VMEM((WIN,), jnp.int32),
        pltpu.VMEM((WIN,), jnp.int32),
        pltpu.VMEM((TWIN,), jnp.int32),
        pltpu.VMEM((WIN, H), jnp.float32),
        pltpu.VMEM_SHARED((NPAD, H), jnp.float32),
        pltpu.SemaphoreType.DMA,
        pltpu.SemaphoreType.DMA,
    ],
)
def _deg_kernel(dst_hbm, zeros_hbm, ones_hbm, out_hbm, dst_w0, dst_w1, dst_t,
                ones_v, acc, sem0, sem1):
    cid = lax.axis_index("c")
    sid = lax.axis_index("s")
    wid = cid * NS + sid
    dbuf = (dst_w0, dst_w1)
    sems = (sem0, sem1)
    descs = [None] * NF
    descs[0] = pltpu.async_copy(dst_hbm.at[wid, pl.ds(0, WIN)], dbuf[0], sems[0])
    pltpu.sync_copy(dst_hbm.at[wid, pl.ds(NF * WIN, TWIN)], dst_t)
    _zero_acc(zeros_hbm, acc, sid)
    pltpu.sync_copy(ones_hbm, ones_v)
    plsc.subcore_barrier()
    for w in range(NF):
        b = w % 2
        if w + 1 < NF:
            descs[w + 1] = pltpu.async_copy(
                dst_hbm.at[wid, pl.ds((w + 1) * WIN, WIN)], dbuf[1 - b], sems[1 - b]
            )
        descs[w].wait()
        pltpu.sync_copy(ones_v, acc.at[dbuf[b]], add=True)
    pltpu.sync_copy(ones_v.at[pl.ds(0, TWIN)], acc.at[dst_t], add=True)
    plsc.subcore_barrier()
    pltpu.sync_copy(
        acc.at[pl.ds(sid * OROWS, OROWS)], out_hbm.at[cid, pl.ds(sid * OROWS, OROWS)]
    )

    @pl.when(sid == NS - 1)
    def _tail():
        pltpu.sync_copy(
            acc.at[pl.ds(NS * OROWS, TAIL)], out_hbm.at[cid, pl.ds(NS * OROWS, TAIL)]
        )


@functools.partial(
    pl.kernel,
    out_type=jax.ShapeDtypeStruct((NC, N, H), jnp.float32),
    mesh=_MESH,
    scratch_types=[
        pltpu.VMEM((WIN,), jnp.int32),
        pltpu.VMEM((WIN,), jnp.int32),
        pltpu.VMEM((WIN,), jnp.int32),
        pltpu.VMEM((WIN,), jnp.int32),
        pltpu.VMEM((WIN,), jnp.int32),
        pltpu.VMEM((WIN,), jnp.int32),
        pltpu.VMEM((TWIN,), jnp.int32),
        pltpu.VMEM((TWIN,), jnp.int32),
        pltpu.VMEM((WIN, H), jnp.float32),
        pltpu.VMEM((WIN, H), jnp.float32),
        pltpu.VMEM_SHARED((NPAD, H), jnp.float32),
        pltpu.SemaphoreType.DMA,
        pltpu.SemaphoreType.DMA,
        pltpu.SemaphoreType.DMA,
        pltpu.SemaphoreType.DMA,
        pltpu.SemaphoreType.DMA,
        pltpu.SemaphoreType.DMA,
        pltpu.SemaphoreType.DMA,
        pltpu.SemaphoreType.DMA,
        pltpu.SemaphoreType.DMA,
        pltpu.SemaphoreType.DMA,
    ],
)
def _msg_kernel(hp_hbm, src_hbm, dst_hbm, zeros_hbm, out_hbm,
                src_w0, src_w1, src_w2, dst_w0, dst_w1, dst_w2, src_t, dst_t,
                rows0, rows1, acc, is0, is1, is2, js0, js1, js2,
                gs0, gs1, ss0, ss1):
    cid = lax.axis_index("c")
    sid = lax.axis_index("s")
    wid = cid * NS + sid
    sbuf = (src_w0, src_w1, src_w2)
    dbuf = (dst_w0, dst_w1, dst_w2)
    rbuf = (rows0, rows1)
    isems = (is0, is1, is2)
    jsems = (js0, js1, js2)
    gsems = (gs0, gs1)
    ssems = (ss0, ss1)
    si = [None] * NF
    di = [None] * NF
    gd = [None] * NF
    sc = [None] * NF
    for p in range(2):
        si[p] = pltpu.async_copy(
            src_hbm.at[wid, pl.ds(p * WIN, WIN)], sbuf[p], isems[p]
        )
        di[p] = pltpu.async_copy(
            dst_hbm.at[wid, pl.ds(p * WIN, WIN)], dbuf[p], jsems[p]
        )
    pltpu.sync_copy(src_hbm.at[wid, pl.ds(NF * WIN, TWIN)], src_t)
    pltpu.sync_copy(dst_hbm.at[wid, pl.ds(NF * WIN, TWIN)], dst_t)
    _zero_acc(zeros_hbm, acc, sid)
    plsc.subcore_barrier()
    si[0].wait()
    gd[0] = pltpu.async_copy(hp_hbm.at[sbuf[0]], rbuf[0], gsems[0])
    for w in range(NF):
        b = w % 2
        nb = 1 - b
        gd[w].wait()
        di[w].wait()
        sc[w] = pltpu.async_copy(rbuf[b], acc.at[dbuf[w % 3]], ssems[b],
                                 add=True)
        if w >= 1:
            sc[w - 1].wait()
        if w + 2 < NF:
            s3 = (w + 2) % 3
            si[w + 2] = pltpu.async_copy(
                src_hbm.at[wid, pl.ds((w + 2) * WIN, WIN)], sbuf[s3], isems[s3]
            )
            di[w + 2] = pltpu.async_copy(
                dst_hbm.at[wid, pl.ds((w + 2) * WIN, WIN)], dbuf[s3], jsems[s3]
            )
        if w + 1 < NF:
            si[w + 1].wait()
            gd[w + 1] = pltpu.async_copy(hp_hbm.at[sbuf[(w + 1) % 3]], rbuf[nb],
                                         gsems[nb])
    sc[NF - 1].wait()
    tb = NF % 2
    pltpu.async_copy(hp_hbm.at[src_t], rbuf[tb].at[pl.ds(0, TWIN)],
                     gsems[tb]).wait()
    pltpu.sync_copy(rbuf[tb].at[pl.ds(0, TWIN)], acc.at[dst_t], add=True)
    plsc.subcore_barrier()
    pltpu.sync_copy(
        acc.at[pl.ds(sid * OROWS, OROWS)], out_hbm.at[cid, pl.ds(sid * OROWS, OROWS)]
    )

    @pl.when(sid == NS - 1)
    def _tail():
        pltpu.sync_copy(
            acc.at[pl.ds(NS * OROWS, TAIL)], out_hbm.at[cid, pl.ds(NS * OROWS, TAIL)]
        )


def _prep_body(dp_ref, x_ref, w0_ref, dinv_ref, hp_ref):
    deg = dp_ref[0][:, 0:1] + dp_ref[1][:, 0:1] + 1.0
    dinv = lax.rsqrt(deg)
    dinv_ref[...] = dinv
    h = jnp.dot(x_ref[...], w0_ref[...], preferred_element_type=jnp.float32,
                precision=lax.Precision.HIGHEST)
    hp_ref[...] = h * dinv


def _bn_body(agg_ref, hp_ref, dinv_ref, b_ref, g_ref, be_ref, yprev_ref,
             y_ref, *, has_resid):
    z = (agg_ref[0] + agg_ref[1] + hp_ref[...]) * dinv_ref[...] + b_ref[...]
    if has_resid:
        z = z + yprev_ref[...]
    mu = jnp.mean(z, axis=0, keepdims=True)
    d = z - mu
    var = jnp.mean(d * d, axis=0, keepdims=True)
    y = g_ref[...] * d * lax.rsqrt(var + 1e-5) + be_ref[...]
    y_ref[...] = jnp.maximum(y, 0.0)


def _mm_body(y_ref, wn_ref, dinv_ref, hn_ref):
    hn_ref[...] = jnp.dot(y_ref[...], wn_ref[...],
                          preferred_element_type=jnp.float32,
                          precision=lax.Precision.HIGHEST) * dinv_ref[...]


_F = jnp.float32
_prep_call = pl.pallas_call(
    _prep_body,
    out_shape=(jax.ShapeDtypeStruct((N, 1), _F), jax.ShapeDtypeStruct((N, H), _F)))
_bn0_call = pl.pallas_call(
    functools.partial(_bn_body, has_resid=False),
    out_shape=jax.ShapeDtypeStruct((N, H), _F))
_bn_resid_call = pl.pallas_call(
    functools.partial(_bn_body, has_resid=True),
    out_shape=jax.ShapeDtypeStruct((N, H), _F))
_mm_call = pl.pallas_call(
    _mm_body, out_shape=jax.ShapeDtypeStruct((N, H), _F))


def kernel(x, edge_index, W0, b0, g0, be0, W1, b1, g1, be1, W2, b2, g2, be2):
    src_w = edge_index[0].reshape(NW, EPW)
    dst_w = edge_index[1].reshape(NW, EPW)

    zeros_hbm = jnp.zeros((NPAD, H), _F)
    ones_hbm = jnp.ones((WIN, H), _F)

    b0r, g0r, be0r = b0.reshape(1, H), g0.reshape(1, H), be0.reshape(1, H)
    b1r, g1r, be1r = b1.reshape(1, H), g1.reshape(1, H), be1.reshape(1, H)
    b2r, g2r, be2r = b2.reshape(1, H), g2.reshape(1, H), be2.reshape(1, H)

    dp = _deg_kernel(dst_w, zeros_hbm, ones_hbm)
    dinv, h0p = _prep_call(dp, x, W0)
    agg0 = _msg_kernel(h0p, src_w, dst_w, zeros_hbm)
    y0 = _bn0_call(agg0, h0p, dinv, b0r, g0r, be0r, h0p)
    h1p = _mm_call(y0, W1, dinv)
    agg1 = _msg_kernel(h1p, src_w, dst_w, zeros_hbm)
    y1 = _bn_resid_call(agg1, h1p, dinv, b1r, g1r, be1r, y0)
    h2p = _mm_call(y1, W2, dinv)
    agg2 = _msg_kernel(h2p, src_w, dst_w, zeros_hbm)
    y2 = _bn_resid_call(agg2, h2p, dinv, b2r, g2r, be2r, y1)
    return y2


# speedup vs baseline: 1.0603x; 1.0233x over previous
import functools

import jax
import jax.numpy as jnp
from jax import lax
from jax.experimental import pallas as pl
from jax.experimental.pallas import tpu as pltpu
from jax.experimental.pallas import tpu_sc as plsc

N = 10000
E = 320000
H = 128

NC = 2
NS = 16
NW = NC * NS
WIN = 128
GW = E // WIN
NF = GW // NW
XTRA = GW - NF * NW
NPAD = 10112
ZROWS = NPAD // NS
OROWS = 624
TAIL = N - NS * OROWS

_MESH = plsc.VectorSubcoreMesh(
    core_axis_name="c", subcore_axis_name="s", num_cores=NC, num_subcores=NS
)


def _zero_acc(zeros_hbm, acc, sid):
    pltpu.sync_copy(
        zeros_hbm.at[pl.ds(sid * ZROWS, ZROWS)], acc.at[pl.ds(sid * ZROWS, ZROWS)]
    )


@functools.partial(
    pl.kernel,
    out_type=jax.ShapeDtypeStruct((NC, N, H), jnp.float32),
    mesh=_MESH,
    scratch_types=[
        pltpu.VMEM((WIN,), jnp.int32),
        pltpu.VMEM((WIN,), jnp.int32),
        pltpu.VMEM((WIN,), jnp.int32),
        pltpu.VMEM((WIN, H), jnp.float32),
        pltpu.VMEM_SHARED((NPAD, H), jnp.float32),
        pltpu.SemaphoreType.DMA,
        pltpu.SemaphoreType.DMA,
    ],
)
def _deg_kernel(edge_hbm, zeros_hbm, ones_hbm, out_hbm, dst_w0, dst_w1, dst_t,
                ones_v, acc, sem0, sem1):
    cid = lax.axis_index("c")
    sid = lax.axis_index("s")
    wid = cid * NS + sid
    dbuf = (dst_w0, dst_w1)
    sems = (sem0, sem1)
    descs = [None] * NF
    descs[0] = pltpu.async_copy(
        edge_hbm.at[1, pl.ds(wid * WIN, WIN)], dbuf[0], sems[0]
    )
    _zero_acc(zeros_hbm, acc, sid)
    pltpu.sync_copy(ones_hbm, ones_v)
    plsc.subcore_barrier()
    for w in range(NF):
        b = w % 2
        if w + 1 < NF:
            descs[w + 1] = pltpu.async_copy(
                edge_hbm.at[1, pl.ds((wid + (w + 1) * NW) * WIN, WIN)],
                dbuf[1 - b], sems[1 - b]
            )
        descs[w].wait()
        pltpu.sync_copy(ones_v, acc.at[dbuf[b]], add=True)

    @pl.when(wid < XTRA)
    def _extra():
        pltpu.sync_copy(edge_hbm.at[1, pl.ds((wid + NF * NW) * WIN, WIN)], dst_t)
        pltpu.sync_copy(ones_v, acc.at[dst_t], add=True)

    plsc.subcore_barrier()
    pltpu.sync_copy(
        acc.at[pl.ds(sid * OROWS, OROWS)], out_hbm.at[cid, pl.ds(sid * OROWS, OROWS)]
    )

    @pl.when(sid == NS - 1)
    def _tail():
        pltpu.sync_copy(
            acc.at[pl.ds(NS * OROWS, TAIL)], out_hbm.at[cid, pl.ds(NS * OROWS, TAIL)]
        )


@functools.partial(
    pl.kernel,
    out_type=jax.ShapeDtypeStruct((NC, N, H), jnp.float32),
    mesh=_MESH,
    scratch_types=[
        pltpu.VMEM((WIN,), jnp.int32),
        pltpu.VMEM((WIN,), jnp.int32),
        pltpu.VMEM((WIN,), jnp.int32),
        pltpu.VMEM((WIN,), jnp.int32),
        pltpu.VMEM((WIN,), jnp.int32),
        pltpu.VMEM((WIN,), jnp.int32),
        pltpu.VMEM((WIN,), jnp.int32),
        pltpu.VMEM((WIN,), jnp.int32),
        pltpu.VMEM((WIN, H), jnp.float32),
        pltpu.VMEM((WIN, H), jnp.float32),
        pltpu.VMEM_SHARED((NPAD, H), jnp.float32),
        pltpu.SemaphoreType.DMA,
        pltpu.SemaphoreType.DMA,
        pltpu.SemaphoreType.DMA,
        pltpu.SemaphoreType.DMA,
        pltpu.SemaphoreType.DMA,
        pltpu.SemaphoreType.DMA,
        pltpu.SemaphoreType.DMA,
        pltpu.SemaphoreType.DMA,
        pltpu.SemaphoreType.DMA,
        pltpu.SemaphoreType.DMA,
    ],
)
def _msg_kernel(hp_hbm, edge_hbm, zeros_hbm, out_hbm,
                src_w0, src_w1, src_w2, dst_w0, dst_w1, dst_w2, src_t, dst_t,
                rows0, rows1, acc, is0, is1, is2, js0, js1, js2,
                gs0, gs1, ss0, ss1):
    cid = lax.axis_index("c")
    sid = lax.axis_index("s")
    wid = cid * NS + sid
    sbuf = (src_w0, src_w1, src_w2)
    dbuf = (dst_w0, dst_w1, dst_w2)
    rbuf = (rows0, rows1)
    isems = (is0, is1, is2)
    jsems = (js0, js1, js2)
    gsems = (gs0, gs1)
    ssems = (ss0, ss1)
    si = [None] * NF
    di = [None] * NF
    gd = [None] * NF
    sc = [None] * NF
    for p in range(2):
        si[p] = pltpu.async_copy(
            edge_hbm.at[0, pl.ds((wid + p * NW) * WIN, WIN)], sbuf[p], isems[p]
        )
        di[p] = pltpu.async_copy(
            edge_hbm.at[1, pl.ds((wid + p * NW) * WIN, WIN)], dbuf[p], jsems[p]
        )
    _zero_acc(zeros_hbm, acc, sid)
    plsc.subcore_barrier()
    si[0].wait()
    gd[0] = pltpu.async_copy(hp_hbm.at[sbuf[0]], rbuf[0], gsems[0])
    for w in range(NF):
        b = w % 2
        nb = 1 - b
        gd[w].wait()
        di[w].wait()
        sc[w] = pltpu.async_copy(rbuf[b], acc.at[dbuf[w % 3]], ssems[b],
                                 add=True)
        if w >= 1:
            sc[w - 1].wait()
        if w + 2 < NF:
            s3 = (w + 2) % 3
            si[w + 2] = pltpu.async_copy(
                edge_hbm.at[0, pl.ds((wid + (w + 2) * NW) * WIN, WIN)],
                sbuf[s3], isems[s3]
            )
            di[w + 2] = pltpu.async_copy(
                edge_hbm.at[1, pl.ds((wid + (w + 2) * NW) * WIN, WIN)],
                dbuf[s3], jsems[s3]
            )
        if w + 1 < NF:
            si[w + 1].wait()
            gd[w + 1] = pltpu.async_copy(hp_hbm.at[sbuf[(w + 1) % 3]], rbuf[nb],
                                         gsems[nb])
    sc[NF - 1].wait()

    @pl.when(wid < XTRA)
    def _extra():
        tb = NF % 2
        off = (wid + NF * NW) * WIN
        pltpu.sync_copy(edge_hbm.at[0, pl.ds(off, WIN)], src_t)
        pltpu.sync_copy(edge_hbm.at[1, pl.ds(off, WIN)], dst_t)
        pltpu.async_copy(hp_hbm.at[src_t], rbuf[tb], gsems[tb]).wait()
        pltpu.sync_copy(rbuf[tb], acc.at[dst_t], add=True)

    plsc.subcore_barrier()
    pltpu.sync_copy(
        acc.at[pl.ds(sid * OROWS, OROWS)], out_hbm.at[cid, pl.ds(sid * OROWS, OROWS)]
    )

    @pl.when(sid == NS - 1)
    def _tail():
        pltpu.sync_copy(
            acc.at[pl.ds(NS * OROWS, TAIL)], out_hbm.at[cid, pl.ds(NS * OROWS, TAIL)]
        )


def _prep_body(dp_ref, x_ref, w0_ref, dinv_ref, hp_ref):
    deg = dp_ref[0][:, 0:1] + dp_ref[1][:, 0:1] + 1.0
    dinv = lax.rsqrt(deg)
    dinv_ref[...] = dinv
    h = jnp.dot(x_ref[...], w0_ref[...], preferred_element_type=jnp.float32,
                precision=lax.Precision.HIGHEST)
    hp_ref[...] = h * dinv


def _bn_body(agg_ref, hp_ref, dinv_ref, b_ref, g_ref, be_ref, yprev_ref,
             y_ref, *, has_resid):
    z = (agg_ref[0] + agg_ref[1] + hp_ref[...]) * dinv_ref[...] + b_ref[...]
    if has_resid:
        z = z + yprev_ref[...]
    mu = jnp.mean(z, axis=0, keepdims=True)
    d = z - mu
    var = jnp.mean(d * d, axis=0, keepdims=True)
    y = g_ref[...] * d * lax.rsqrt(var + 1e-5) + be_ref[...]
    y_ref[...] = jnp.maximum(y, 0.0)


def _mm_body(y_ref, wn_ref, dinv_ref, hn_ref):
    hn_ref[...] = jnp.dot(y_ref[...], wn_ref[...],
                          preferred_element_type=jnp.float32,
                          precision=lax.Precision.HIGHEST) * dinv_ref[...]


_F = jnp.float32
_prep_call = pl.pallas_call(
    _prep_body,
    out_shape=(jax.ShapeDtypeStruct((N, 1), _F), jax.ShapeDtypeStruct((N, H), _F)))
_bn0_call = pl.pallas_call(
    functools.partial(_bn_body, has_resid=False),
    out_shape=jax.ShapeDtypeStruct((N, H), _F))
_bn_resid_call = pl.pallas_call(
    functools.partial(_bn_body, has_resid=True),
    out_shape=jax.ShapeDtypeStruct((N, H), _F))
_mm_call = pl.pallas_call(
    _mm_body, out_shape=jax.ShapeDtypeStruct((N, H), _F))


def kernel(x, edge_index, W0, b0, g0, be0, W1, b1, g1, be1, W2, b2, g2, be2):
    zeros_hbm = jnp.zeros((NPAD, H), _F)
    ones_hbm = jnp.ones((WIN, H), _F)

    b0r, g0r, be0r = b0.reshape(1, H), g0.reshape(1, H), be0.reshape(1, H)
    b1r, g1r, be1r = b1.reshape(1, H), g1.reshape(1, H), be1.reshape(1, H)
    b2r, g2r, be2r = b2.reshape(1, H), g2.reshape(1, H), be2.reshape(1, H)

    dp = _deg_kernel(edge_index, zeros_hbm, ones_hbm)
    dinv, h0p = _prep_call(dp, x, W0)
    agg0 = _msg_kernel(h0p, edge_index, zeros_hbm)
    y0 = _bn0_call(agg0, h0p, dinv, b0r, g0r, be0r, h0p)
    h1p = _mm_call(y0, W1, dinv)
    agg1 = _msg_kernel(h1p, edge_index, zeros_hbm)
    y1 = _bn_resid_call(agg1, h1p, dinv, b1r, g1r, be1r, y0)
    h2p = _mm_call(y1, W2, dinv)
    agg2 = _msg_kernel(h2p, edge_index, zeros_hbm)
    y2 = _bn_resid_call(agg2, h2p, dinv, b2r, g2r, be2r, y1)
    return y2


# speedup vs baseline: 1.0652x; 1.0047x over previous
import functools

import jax
import jax.numpy as jnp
from jax import lax
from jax.experimental import pallas as pl
from jax.experimental.pallas import tpu as pltpu
from jax.experimental.pallas import tpu_sc as plsc

N = 10000
E = 320000
H = 128

NC = 2
NS = 16
NW = NC * NS
WIN = 128
GW = E // WIN
NF = GW // NW
XTRA = GW - NF * NW
NPAD = 10112
ZROWS = NPAD // NS
OROWS = 624
TAIL = N - NS * OROWS

_MESH = plsc.VectorSubcoreMesh(
    core_axis_name="c", subcore_axis_name="s", num_cores=NC, num_subcores=NS
)


def _zero_acc(zeros_hbm, acc, sid):
    pltpu.sync_copy(
        zeros_hbm.at[pl.ds(sid * ZROWS, ZROWS)], acc.at[pl.ds(sid * ZROWS, ZROWS)]
    )


@functools.partial(
    pl.kernel,
    out_type=jax.ShapeDtypeStruct((NC, N, H), jnp.float32),
    mesh=_MESH,
    scratch_types=[
        pltpu.VMEM((WIN,), jnp.int32),
        pltpu.VMEM((WIN,), jnp.int32),
        pltpu.VMEM((WIN,), jnp.int32),
        pltpu.VMEM((WIN, H), jnp.float32),
        pltpu.VMEM_SHARED((NPAD, H), jnp.float32),
        pltpu.SemaphoreType.DMA,
        pltpu.SemaphoreType.DMA,
    ],
)
def _deg_kernel(edge_hbm, zeros_hbm, ones_hbm, out_hbm, dst_w0, dst_w1, dst_t,
                ones_v, acc, sem0, sem1):
    cid = lax.axis_index("c")
    sid = lax.axis_index("s")
    wid = cid * NS + sid
    dbuf = (dst_w0, dst_w1)
    sems = (sem0, sem1)
    descs = [None] * NF
    descs[0] = pltpu.async_copy(
        edge_hbm.at[1, pl.ds(wid * WIN, WIN)], dbuf[0], sems[0]
    )
    _zero_acc(zeros_hbm, acc, sid)
    pltpu.sync_copy(ones_hbm.at[pl.ds(sid * WIN, WIN)], ones_v)
    plsc.subcore_barrier()
    for w in range(NF):
        b = w % 2
        if w + 1 < NF:
            descs[w + 1] = pltpu.async_copy(
                edge_hbm.at[1, pl.ds((wid + (w + 1) * NW) * WIN, WIN)],
                dbuf[1 - b], sems[1 - b]
            )
        descs[w].wait()
        pltpu.sync_copy(ones_v, acc.at[dbuf[b]], add=True)

    @pl.when(wid < XTRA)
    def _extra():
        pltpu.sync_copy(edge_hbm.at[1, pl.ds((wid + NF * NW) * WIN, WIN)], dst_t)
        pltpu.sync_copy(ones_v, acc.at[dst_t], add=True)

    plsc.subcore_barrier()
    pltpu.sync_copy(
        acc.at[pl.ds(sid * OROWS, OROWS)], out_hbm.at[cid, pl.ds(sid * OROWS, OROWS)]
    )

    @pl.when(sid == NS - 1)
    def _tail():
        pltpu.sync_copy(
            acc.at[pl.ds(NS * OROWS, TAIL)], out_hbm.at[cid, pl.ds(NS * OROWS, TAIL)]
        )


@functools.partial(
    pl.kernel,
    out_type=jax.ShapeDtypeStruct((NC, N, H), jnp.float32),
    mesh=_MESH,
    scratch_types=[
        pltpu.VMEM((WIN,), jnp.int32),
        pltpu.VMEM((WIN,), jnp.int32),
        pltpu.VMEM((WIN,), jnp.int32),
        pltpu.VMEM((WIN,), jnp.int32),
        pltpu.VMEM((WIN,), jnp.int32),
        pltpu.VMEM((WIN,), jnp.int32),
        pltpu.VMEM((WIN,), jnp.int32),
        pltpu.VMEM((WIN,), jnp.int32),
        pltpu.VMEM((WIN, H), jnp.float32),
        pltpu.VMEM((WIN, H), jnp.float32),
        pltpu.VMEM_SHARED((NPAD, H), jnp.float32),
        pltpu.SemaphoreType.DMA,
        pltpu.SemaphoreType.DMA,
        pltpu.SemaphoreType.DMA,
        pltpu.SemaphoreType.DMA,
        pltpu.SemaphoreType.DMA,
        pltpu.SemaphoreType.DMA,
        pltpu.SemaphoreType.DMA,
        pltpu.SemaphoreType.DMA,
        pltpu.SemaphoreType.DMA,
        pltpu.SemaphoreType.DMA,
    ],
)
def _msg_kernel(hp_hbm, edge_hbm, zeros_hbm, out_hbm,
                src_w0, src_w1, src_w2, dst_w0, dst_w1, dst_w2, src_t, dst_t,
                rows0, rows1, acc, is0, is1, is2, js0, js1, js2,
                gs0, gs1, ss0, ss1):
    cid = lax.axis_index("c")
    sid = lax.axis_index("s")
    wid = cid * NS + sid
    sbuf = (src_w0, src_w1, src_w2)
    dbuf = (dst_w0, dst_w1, dst_w2)
    rbuf = (rows0, rows1)
    isems = (is0, is1, is2)
    jsems = (js0, js1, js2)
    gsems = (gs0, gs1)
    ssems = (ss0, ss1)
    si = [None] * NF
    di = [None] * NF
    gd = [None] * NF
    sc = [None] * NF
    for p in range(2):
        si[p] = pltpu.async_copy(
            edge_hbm.at[0, pl.ds((wid + p * NW) * WIN, WIN)], sbuf[p], isems[p]
        )
        di[p] = pltpu.async_copy(
            edge_hbm.at[1, pl.ds((wid + p * NW) * WIN, WIN)], dbuf[p], jsems[p]
        )
    _zero_acc(zeros_hbm, acc, sid)
    plsc.subcore_barrier()
    si[0].wait()
    gd[0] = pltpu.async_copy(hp_hbm.at[sbuf[0]], rbuf[0], gsems[0])
    for w in range(NF):
        b = w % 2
        nb = 1 - b
        gd[w].wait()
        di[w].wait()
        sc[w] = pltpu.async_copy(rbuf[b], acc.at[dbuf[w % 3]], ssems[b],
                                 add=True)
        if w >= 1:
            sc[w - 1].wait()
        if w + 2 < NF:
            s3 = (w + 2) % 3
            si[w + 2] = pltpu.async_copy(
                edge_hbm.at[0, pl.ds((wid + (w + 2) * NW) * WIN, WIN)],
                sbuf[s3], isems[s3]
            )
            di[w + 2] = pltpu.async_copy(
                edge_hbm.at[1, pl.ds((wid + (w + 2) * NW) * WIN, WIN)],
                dbuf[s3], jsems[s3]
            )
        if w + 1 < NF:
            si[w + 1].wait()
            gd[w + 1] = pltpu.async_copy(hp_hbm.at[sbuf[(w + 1) % 3]], rbuf[nb],
                                         gsems[nb])
    sc[NF - 1].wait()

    @pl.when(wid < XTRA)
    def _extra():
        tb = NF % 2
        off = (wid + NF * NW) * WIN
        pltpu.sync_copy(edge_hbm.at[0, pl.ds(off, WIN)], src_t)
        pltpu.sync_copy(edge_hbm.at[1, pl.ds(off, WIN)], dst_t)
        pltpu.async_copy(hp_hbm.at[src_t], rbuf[tb], gsems[tb]).wait()
        pltpu.sync_copy(rbuf[tb], acc.at[dst_t], add=True)

    plsc.subcore_barrier()
    pltpu.sync_copy(
        acc.at[pl.ds(sid * OROWS, OROWS)], out_hbm.at[cid, pl.ds(sid * OROWS, OROWS)]
    )

    @pl.when(sid == NS - 1)
    def _tail():
        pltpu.sync_copy(
            acc.at[pl.ds(NS * OROWS, TAIL)], out_hbm.at[cid, pl.ds(NS * OROWS, TAIL)]
        )


def _prep_body(dp_ref, x_ref, w0_ref, dinv_ref, hp_ref):
    deg = dp_ref[0] + dp_ref[1] + 1.0
    dinv = lax.rsqrt(deg)
    dinv_ref[...] = dinv
    h = jnp.dot(x_ref[...], w0_ref[...], preferred_element_type=jnp.float32,
                precision=lax.Precision.HIGHEST)
    hp_ref[...] = h * dinv


def _bn_body(agg_ref, hp_ref, dinv_ref, b_ref, g_ref, be_ref, yprev_ref,
             y_ref, *, has_resid):
    z = (agg_ref[0] + agg_ref[1] + hp_ref[...]) * dinv_ref[...] + b_ref[...]
    if has_resid:
        z = z + yprev_ref[...]
    mu = jnp.mean(z, axis=0, keepdims=True)
    d = z - mu
    var = jnp.mean(d * d, axis=0, keepdims=True)
    y = g_ref[...] * d * lax.rsqrt(var + 1e-5) + be_ref[...]
    y_ref[...] = jnp.maximum(y, 0.0)


def _mm_body(y_ref, wn_ref, dinv_ref, hn_ref):
    hn_ref[...] = jnp.dot(y_ref[...], wn_ref[...],
                          preferred_element_type=jnp.float32,
                          precision=lax.Precision.HIGHEST) * dinv_ref[...]


_F = jnp.float32
_prep_call = pl.pallas_call(
    _prep_body,
    out_shape=(jax.ShapeDtypeStruct((N, 1), _F), jax.ShapeDtypeStruct((N, H), _F)))
_bn0_call = pl.pallas_call(
    functools.partial(_bn_body, has_resid=False),
    out_shape=jax.ShapeDtypeStruct((N, H), _F))
_bn_resid_call = pl.pallas_call(
    functools.partial(_bn_body, has_resid=True),
    out_shape=jax.ShapeDtypeStruct((N, H), _F))
_mm_call = pl.pallas_call(
    _mm_body, out_shape=jax.ShapeDtypeStruct((N, H), _F))


def kernel(x, edge_index, W0, b0, g0, be0, W1, b1, g1, be1, W2, b2, g2, be2):
    zeros_hbm = jnp.zeros((NPAD, H), _F)
    ones_hbm = jnp.ones((NS * WIN, H), _F)

    b0r, g0r, be0r = b0.reshape(1, H), g0.reshape(1, H), be0.reshape(1, H)
    b1r, g1r, be1r = b1.reshape(1, H), g1.reshape(1, H), be1.reshape(1, H)
    b2r, g2r, be2r = b2.reshape(1, H), g2.reshape(1, H), be2.reshape(1, H)

    dp = _deg_kernel(edge_index, zeros_hbm, ones_hbm)
    dinv, h0p = _prep_call(dp[:, :, 0:1], x, W0)
    agg0 = _msg_kernel(h0p, edge_index, zeros_hbm)
    y0 = _bn0_call(agg0, h0p, dinv, b0r, g0r, be0r, h0p)
    h1p = _mm_call(y0, W1, dinv)
    agg1 = _msg_kernel(h1p, edge_index, zeros_hbm)
    y1 = _bn_resid_call(agg1, h1p, dinv, b1r, g1r, be1r, y0)
    h2p = _mm_call(y1, W2, dinv)
    agg2 = _msg_kernel(h2p, edge_index, zeros_hbm)
    y2 = _bn_resid_call(agg2, h2p, dinv, b2r, g2r, be2r, y1)
    return y2
